# baseline (speedup 1.0000x reference)
import functools

import jax
import jax.numpy as jnp
from jax import lax
from jax.experimental import pallas as pl
from jax.experimental.pallas import tpu as pltpu
from jax.experimental.pallas import tpu_sc as plsc


@functools.lru_cache(maxsize=None)
def _make_gather(N, D, NW, n_ch, CH):
    mesh = plsc.VectorSubcoreMesh(core_axis_name="c", subcore_axis_name="s")
    info = plsc.get_sparse_core_info()
    NC = info.num_cores

    @functools.partial(
        pl.kernel,
        mesh=mesh,
        out_type=jax.ShapeDtypeStruct((N, D), jnp.float32),
        compiler_params=pltpu.CompilerParams(use_tc_tiling_on_sc=False),
        scratch_types=[
            pltpu.VMEM((n_ch, CH), jnp.int32),
            pltpu.VMEM((CH, D), jnp.float32),
            pltpu.SemaphoreType.DMA,
        ],
    )
    def k(ids_hbm, table_hbm, out_hbm, idx_v, rows_v, gsem):
        cid = lax.axis_index("c")
        sid = lax.axis_index("s")
        wid = sid * NC + cid
        base = wid * (n_ch * CH)
        pltpu.sync_copy(ids_hbm.at[wid], idx_v)

        def body(j, carry):
            pltpu.async_copy(table_hbm.at[idx_v.at[j]], rows_v, gsem).wait()
            pltpu.sync_copy(rows_v, out_hbm.at[pl.ds(base + j * CH, CH)])
            return carry

        lax.fori_loop(0, n_ch, body, 0)

    return k


def kernel(input_ids, input_mask, emb_weight):
    B, S = input_ids.shape
    V, D = emb_weight.shape
    N = B * S
    NW = 32
    CH = 128
    n_ch = N // (NW * CH)
    assert N == NW * n_ch * CH
    ids = input_ids.reshape(NW, n_ch, CH)
    out = _make_gather(N, D, NW, n_ch, CH)(ids, emb_weight)
    return (out.reshape(B, S, D), input_mask)


# speedup vs baseline: 1.1078x; 1.1078x over previous
import functools

import jax
import jax.numpy as jnp
from jax import lax
from jax.experimental import pallas as pl
from jax.experimental.pallas import tpu as pltpu
from jax.experimental.pallas import tpu_sc as plsc


@functools.lru_cache(maxsize=None)
def _make_gather(N, D, NW, n_ch, CH, NBUF):
    mesh = plsc.VectorSubcoreMesh(core_axis_name="c", subcore_axis_name="s")
    info = plsc.get_sparse_core_info()
    NC = info.num_cores
    n_grp = n_ch // NBUF
    assert n_grp * NBUF == n_ch and n_grp % 2 == 0 and n_grp >= 4
    n_pairs = n_grp // 2

    @functools.partial(
        pl.kernel,
        mesh=mesh,
        out_type=jax.ShapeDtypeStruct((N, D), jnp.float32),
        compiler_params=pltpu.CompilerParams(use_tc_tiling_on_sc=False),
        scratch_types=[
            pltpu.VMEM((n_ch, CH), jnp.int32),
            pltpu.VMEM((2, NBUF, CH, D), jnp.float32),
            pltpu.SemaphoreType.DMA,
            pltpu.SemaphoreType.DMA,
            pltpu.SemaphoreType.DMA,
            pltpu.SemaphoreType.DMA,
        ],
    )
    def k(ids_hbm, table_hbm, out_hbm, idx_v, rows, gsA, gsB, ssA, ssB):
        cid = lax.axis_index("c")
        sid = lax.axis_index("s")
        wid = sid * NC + cid
        base = wid * (n_ch * CH)
        pltpu.sync_copy(ids_hbm.at[wid], idx_v)
        gsem = (gsA, gsB)
        ssem = (ssA, ssB)

        def g_start(s, b, j):
            pltpu.async_copy(table_hbm.at[idx_v.at[j]], rows.at[s, b], gsem[s])

        def g_wait(s, b):
            pltpu.make_async_copy(
                table_hbm.at[pl.ds(0, CH)], rows.at[s, b], gsem[s]
            ).wait()

        def s_start(s, b, j):
            pltpu.async_copy(
                rows.at[s, b], out_hbm.at[pl.ds(base + j * CH, CH)], ssem[s]
            )

        def s_wait(s, b):
            pltpu.make_async_copy(
                rows.at[s, b], out_hbm.at[pl.ds(base, CH)], ssem[s]
            ).wait()

        for b in range(NBUF):
            g_start(0, b, b)

        for b in range(NBUF):
            g_wait(0, b)
        for b in range(NBUF):
            g_start(1, b, NBUF + b)
        for b in range(NBUF):
            s_start(0, b, b)
        for b in range(NBUF):
            g_wait(1, b)
        for b in range(NBUF):
            s_wait(0, b)
        for b in range(NBUF):
            g_start(0, b, 2 * NBUF + b)
        for b in range(NBUF):
            s_start(1, b, NBUF + b)

        def pair_body(p, carry):
            g0 = 2 * p
            for b in range(NBUF):
                g_wait(0, b)
            for b in range(NBUF):
                s_wait(1, b)
            for b in range(NBUF):
                g_start(1, b, (g0 + 1) * NBUF + b)
            for b in range(NBUF):
                s_start(0, b, g0 * NBUF + b)
            for b in range(NBUF):
                g_wait(1, b)
            for b in range(NBUF):
                s_wait(0, b)
            for b in range(NBUF):
                g_start(0, b, (g0 + 2) * NBUF + b)
            for b in range(NBUF):
                s_start(1, b, (g0 + 1) * NBUF + b)
            return carry

        lax.fori_loop(1, n_pairs - 1, pair_body, 0)

        g0 = n_grp - 2
        for b in range(NBUF):
            g_wait(0, b)
        for b in range(NBUF):
            s_wait(1, b)
        for b in range(NBUF):
            g_start(1, b, (g0 + 1) * NBUF + b)
        for b in range(NBUF):
            s_start(0, b, g0 * NBUF + b)
        for b in range(NBUF):
            g_wait(1, b)
        for b in range(NBUF):
            s_wait(0, b)
        for b in range(NBUF):
            s_start(1, b, (g0 + 1) * NBUF + b)
        for b in range(NBUF):
            s_wait(1, b)

    return k


def kernel(input_ids, input_mask, emb_weight):
    B, S = input_ids.shape
    V, D = emb_weight.shape
    N = B * S
    NW = 32
    CH = 128
    NBUF = 4
    n_ch = N // (NW * CH)
    assert N == NW * n_ch * CH
    ids = input_ids.reshape(NW, n_ch, CH)
    out = _make_gather(N, D, NW, n_ch, CH, NBUF)(ids, emb_weight)
    return (out.reshape(B, S, D), input_mask)


# speedup vs baseline: 1.5829x; 1.4289x over previous
import functools

import jax
import jax.numpy as jnp
from jax import lax
from jax.experimental import pallas as pl
from jax.experimental.pallas import tpu as pltpu
from jax.experimental.pallas import tpu_sc as plsc


@functools.lru_cache(maxsize=None)
def _make_gather(V2, D, NW, n_ch, CH, NBUF):
    mesh = plsc.VectorSubcoreMesh(core_axis_name="c", subcore_axis_name="s")
    info = plsc.get_sparse_core_info()
    NC = info.num_cores
    n_grp = n_ch // NBUF
    assert n_grp * NBUF == n_ch and n_grp % 2 == 0 and n_grp >= 6
    n_pairs = n_grp // 2
    n_chunks = NW * n_ch

    @functools.partial(
        pl.kernel,
        mesh=mesh,
        out_type=jax.ShapeDtypeStruct((n_chunks, CH, 2 * D), jnp.float32),
        compiler_params=pltpu.CompilerParams(use_tc_tiling_on_sc=False),
        scratch_types=[
            pltpu.VMEM((n_ch, CH), jnp.int32),
            pltpu.VMEM((2, NBUF, CH, D), jnp.float32),
            pltpu.SemaphoreType.DMA,
            pltpu.SemaphoreType.DMA,
            pltpu.SemaphoreType.DMA,
            pltpu.SemaphoreType.DMA,
        ],
    )
    def k(ids_hbm, table_hbm, out_hbm, idx_v, rows, gsA, gsB, ssA, ssB):
        cid = lax.axis_index("c")
        sid = lax.axis_index("s")
        wid = sid * NC + cid
        base = wid * n_ch
        pltpu.sync_copy(ids_hbm.at[wid], idx_v)
        gsem = (gsA, gsB)
        ssem = (ssA, ssB)

        def g_start(s, b, j):
            pltpu.async_copy(table_hbm.at[idx_v.at[j]], rows.at[s, b], gsem[s])

        def g_wait(s, b):
            pltpu.make_async_copy(
                table_hbm.at[pl.ds(0, CH)], rows.at[s, b], gsem[s]
            ).wait()

        def s_start(s, b, j):
            pltpu.async_copy(
                rows.at[s, b], out_hbm.at[base + j, :, pl.ds(0, D)], ssem[s]
            )

        def s_wait(s, b):
            pltpu.make_async_copy(
                rows.at[s, b], out_hbm.at[base, :, pl.ds(0, D)], ssem[s]
            ).wait()

        for b in range(NBUF):
            g_start(0, b, b)

        for b in range(NBUF):
            g_wait(0, b)
        for b in range(NBUF):
            g_start(1, b, NBUF + b)
        for b in range(NBUF):
            s_start(0, b, b)
        for b in range(NBUF):
            g_wait(1, b)
        for b in range(NBUF):
            s_wait(0, b)
        for b in range(NBUF):
            g_start(0, b, 2 * NBUF + b)
        for b in range(NBUF):
            s_start(1, b, NBUF + b)

        def pair_body(p, carry):
            g0 = 2 * p
            for b in range(NBUF):
                g_wait(0, b)
            for b in range(NBUF):
                s_wait(1, b)
            for b in range(NBUF):
                g_start(1, b, (g0 + 1) * NBUF + b)
            for b in range(NBUF):
                s_start(0, b, g0 * NBUF + b)
            for b in range(NBUF):
                g_wait(1, b)
            for b in range(NBUF):
                s_wait(0, b)
            for b in range(NBUF):
                g_start(0, b, (g0 + 2) * NBUF + b)
            for b in range(NBUF):
                s_start(1, b, (g0 + 1) * NBUF + b)
            return carry

        lax.fori_loop(1, n_pairs - 1, pair_body, 0)

        g0 = n_grp - 2
        for b in range(NBUF):
            g_wait(0, b)
        for b in range(NBUF):
            s_wait(1, b)
        for b in range(NBUF):
            g_start(1, b, (g0 + 1) * NBUF + b)
        for b in range(NBUF):
            s_start(0, b, g0 * NBUF + b)
        for b in range(NBUF):
            g_wait(1, b)
        for b in range(NBUF):
            s_wait(0, b)
        for b in range(NBUF):
            s_start(1, b, (g0 + 1) * NBUF + b)
        for b in range(NBUF):
            s_wait(1, b)

    return k


def kernel(input_ids, input_mask, emb_weight):
    B, S = input_ids.shape
    V, D = emb_weight.shape
    N = B * S
    NW = 32
    CH = 128
    NBUF = 4
    n_ch = N // (NW * CH)
    assert N == NW * n_ch * CH
    t2 = jnp.pad(emb_weight, ((0, 0), (0, D))).reshape(2 * V, D)
    ids2 = (input_ids.reshape(N) * 2).reshape(NW, n_ch, CH)
    out3 = _make_gather(2 * V, D, NW, n_ch, CH, NBUF)(ids2, t2)
    out = out3.reshape(N, 2 * D)[:, :D].reshape(B, S, D)
    return (out, input_mask)
